# Initial kernel scaffold; baseline (speedup 1.0000x reference)
import jax
import jax.numpy as jnp
from jax.experimental import pallas as pl


def kernel(x, edge_index, W1_l, b1_l, W1_r, W2_l, b2_l, W2_r, Wc, bc):
    raise NotImplementedError("write your pallas kernel here")


# speedup vs baseline: 3.4474x; 3.4474x over previous
import functools

import jax
import jax.numpy as jnp
from jax import lax
from jax.experimental import pallas as pl
from jax.experimental.pallas import tpu as pltpu
from jax.experimental.pallas import tpu_sc as plsc

N = 10000
E = 160000
D = 256
H = 512
C = 16

NP = 10240
W = 128
NWIN = E // W
NSUB = 16
ROWS_PER_TILE = NP // NSUB
RB = 512
GRID = NP // RB


def _sc_agg_body(K, with_deg, *refs):
    if with_deg:
        (table, src_hbm, dst_hbm, agg_out, deg_out,
         acc, deg_acc, srcv, dstv, gidxv, rows, zbuf, onesv, sem) = refs
    else:
        (table, src_hbm, dst_hbm, agg_out,
         acc, deg_acc, srcv, dstv, gidxv, rows, zbuf, onesv, sem) = refs
        deg_out = None

    cid = lax.axis_index("c")
    sid = lax.axis_index("s")
    passes = K // 2

    zeros16 = jnp.zeros((16,), jnp.float32)
    ones16 = jnp.ones((16,), jnp.float32)

    @pl.loop(0, W)
    def _(r):
        @pl.loop(0, 128, step=16)
        def _(i):
            zbuf[r, pl.ds(i, 16)] = zeros16

    @pl.loop(0, W, step=16)
    def _(i):
        onesv[pl.ds(i, 16)] = ones16

    def zero_acc():
        @pl.loop(0, ROWS_PER_TILE // W)
        def _(j):
            off = sid * ROWS_PER_TILE + j * W
            pltpu.sync_copy(zbuf, acc.at[pl.ds(off, W)])

    zero_acc()
    if with_deg:
        @pl.when(cid == 0)
        def _():
            @pl.loop(0, ROWS_PER_TILE // W)
            def _(j):
                off = sid * ROWS_PER_TILE + j * W
                pltpu.sync_copy(zbuf.at[0], deg_acc.at[pl.ds(off, W)])
    plsc.subcore_barrier()

    for kk in range(passes):
        chunk = cid * passes + kk

        @pl.loop(sid, NWIN, step=NSUB)
        def _(w):
            base = w * W
            pltpu.sync_copy(src_hbm.at[pl.ds(base, W)], srcv)
            pltpu.sync_copy(dst_hbm.at[pl.ds(base, W)], dstv)

            @pl.loop(0, W, step=16)
            def _(i):
                gidxv[pl.ds(i, 16)] = srcv[pl.ds(i, 16)] * K + chunk

            pltpu.async_copy(table.at[gidxv], rows, sem).wait()
            pltpu.sync_copy(rows, acc.at[dstv], add=True)
            if with_deg and kk == 0:
                @pl.when(cid == 0)
                def _():
                    pltpu.sync_copy(onesv, deg_acc.at[dstv], add=True)

        plsc.subcore_barrier()

        @pl.loop(0, ROWS_PER_TILE // W)
        def _(j):
            off = sid * ROWS_PER_TILE + j * W
            pltpu.sync_copy(acc.at[pl.ds(off, W)],
                            agg_out.at[chunk, pl.ds(off, W)])
        if with_deg and kk == 0:
            @pl.when(cid == 0)
            def _():
                @pl.loop(0, ROWS_PER_TILE // W)
                def _(j):
                    off = sid * ROWS_PER_TILE + j * W
                    pltpu.sync_copy(deg_acc.at[pl.ds(off, W)],
                                    deg_out.at[pl.ds(off, W)])

        if kk + 1 < passes:
            zero_acc()
            plsc.subcore_barrier()


def _sc_agg(table, src, dst, K, with_deg):
    mesh = plsc.VectorSubcoreMesh(core_axis_name="c", subcore_axis_name="s")
    out_type = [jax.ShapeDtypeStruct((K, NP, 128), jnp.float32)]
    if with_deg:
        out_type.append(jax.ShapeDtypeStruct((NP,), jnp.float32))
    scratch = [
        pltpu.VMEM_SHARED((NP, 128), jnp.float32),
        pltpu.VMEM_SHARED((NP,), jnp.float32),
        pltpu.VMEM((W,), jnp.int32),
        pltpu.VMEM((W,), jnp.int32),
        pltpu.VMEM((W,), jnp.int32),
        pltpu.VMEM((W, 128), jnp.float32),
        pltpu.VMEM((W, 128), jnp.float32),
        pltpu.VMEM((W,), jnp.float32),
        pltpu.SemaphoreType.DMA,
    ]
    body = functools.partial(_sc_agg_body, K, with_deg)
    fn = pl.kernel(body, out_type=out_type, mesh=mesh, scratch_types=scratch,
                   name=f"sc_sage_agg_k{K}")
    return fn(table, src, dst)


_DOT = functools.partial(
    lax.dot_general,
    dimension_numbers=(((1,), (1,)), ((), ())),
    precision=lax.Precision.HIGHEST,
    preferred_element_type=jnp.float32,
)


def _tc_lin_r_body(x_ref, w_ref, o_ref):
    o_ref[...] = _DOT(x_ref[...], w_ref[...])


def _tc_lin_r(x, w):
    f = x.shape[1]
    h = w.shape[0]
    return pl.pallas_call(
        _tc_lin_r_body,
        grid=(GRID,),
        in_specs=[pl.BlockSpec((RB, f), lambda i: (i, 0)),
                  pl.BlockSpec((h, f), lambda i: (0, 0))],
        out_specs=pl.BlockSpec((RB, h), lambda i: (i, 0)),
        out_shape=jax.ShapeDtypeStruct((NP, h), jnp.float32),
    )(x, w)


def _tc_layer_body(nchunks, emit_next, agg_ref, recip_ref, xr_ref, wl_ref,
                   b_ref, *rest):
    if emit_next:
        wn_ref, h_ref, xrn_ref = rest
    else:
        wc_ref, bc_ref, o_ref = rest
    z = _DOT(agg_ref[0], wl_ref[:, pl.ds(0, 128)])
    for k in range(1, nchunks):
        z += _DOT(agg_ref[k], wl_ref[:, pl.ds(k * 128, 128)])
    z = z * recip_ref[...] + xr_ref[...] + b_ref[...]
    h = jnp.maximum(z, 0.0)
    if emit_next:
        h_ref[...] = h
        xrn_ref[...] = _DOT(h, wn_ref[...])
    else:
        o_ref[...] = _DOT(h, wc_ref[...]) + bc_ref[...]


def _tc_layer1(agg, recip, xr, wl, b, wn):
    return pl.pallas_call(
        functools.partial(_tc_layer_body, 2, True),
        grid=(GRID,),
        in_specs=[pl.BlockSpec((2, RB, 128), lambda i: (0, i, 0)),
                  pl.BlockSpec((RB, 1), lambda i: (i, 0)),
                  pl.BlockSpec((RB, H), lambda i: (i, 0)),
                  pl.BlockSpec((H, D), lambda i: (0, 0)),
                  pl.BlockSpec((1, H), lambda i: (0, 0)),
                  pl.BlockSpec((H, H), lambda i: (0, 0))],
        out_specs=[pl.BlockSpec((RB, H), lambda i: (i, 0)),
                   pl.BlockSpec((RB, H), lambda i: (i, 0))],
        out_shape=[jax.ShapeDtypeStruct((NP, H), jnp.float32),
                   jax.ShapeDtypeStruct((NP, H), jnp.float32)],
    )(agg, recip, xr, wl, b, wn)


def _tc_layer2(agg, recip, xr, wl, b, wc, bc):
    return pl.pallas_call(
        functools.partial(_tc_layer_body, 4, False),
        grid=(GRID,),
        in_specs=[pl.BlockSpec((4, RB, 128), lambda i: (0, i, 0)),
                  pl.BlockSpec((RB, 1), lambda i: (i, 0)),
                  pl.BlockSpec((RB, H), lambda i: (i, 0)),
                  pl.BlockSpec((H, H), lambda i: (0, 0)),
                  pl.BlockSpec((1, H), lambda i: (0, 0)),
                  pl.BlockSpec((C, H), lambda i: (0, 0)),
                  pl.BlockSpec((1, C), lambda i: (0, 0))],
        out_specs=pl.BlockSpec((RB, C), lambda i: (i, 0)),
        out_shape=jax.ShapeDtypeStruct((NP, C), jnp.float32),
    )(agg, recip, xr, wl, b, wc, bc)


def kernel(x, edge_index, W1_l, b1_l, W1_r, W2_l, b2_l, W2_r, Wc, bc):
    src = edge_index[0]
    dst = edge_index[1]
    x_p = jnp.zeros((NP, D), jnp.float32).at[:N].set(x)

    agg1, deg = _sc_agg(x_p.reshape(2 * NP, 128), src, dst, 2, True)
    recip = (1.0 / jnp.clip(deg, 1.0)).reshape(NP, 1)

    xr1 = _tc_lin_r(x_p, W1_r)
    h1, xr2 = _tc_layer1(agg1, recip, xr1, W1_l, b1_l.reshape(1, H), W2_r)

    (agg2,) = _sc_agg(h1.reshape(4 * NP, 128), src, dst, 4, False)

    out = _tc_layer2(agg2, recip, xr2, W2_l, b2_l.reshape(1, H),
                     Wc, bc.reshape(1, C))
    return out[:N]


# speedup vs baseline: 4.8860x; 1.4173x over previous
import functools

import jax
import jax.numpy as jnp
from jax import lax
from jax.experimental import pallas as pl
from jax.experimental.pallas import tpu as pltpu
from jax.experimental.pallas import tpu_sc as plsc

N = 10000
E = 160000
D = 256
H = 512
C = 16

NP = 10240
W = 128
NWIN = E // W
NSUB = 16
ROWS_PER_TILE = NP // NSUB
RB = 512
GRID = NP // RB


def _sc_agg_body(K, with_deg, *refs):
    if with_deg:
        (table, src_hbm, dst_hbm, agg_out, deg_out,
         acc, deg_acc, srcva, dstva, gidxva, srcvb, dstvb, gidxvb,
         rowsa, rowsb, onesv, sema, semb) = refs
    else:
        (table, src_hbm, dst_hbm, agg_out,
         acc, deg_acc, srcva, dstva, gidxva, srcvb, dstvb, gidxvb,
         rowsa, rowsb, onesv, sema, semb) = refs
        deg_out = None

    cid = lax.axis_index("c")
    sid = lax.axis_index("s")
    passes = K // 2

    zeros16 = jnp.zeros((16,), jnp.float32)
    ones16 = jnp.ones((16,), jnp.float32)

    @pl.loop(0, W, step=16)
    def _(i):
        onesv[pl.ds(i, 16)] = ones16

    def zero_acc():
        @pl.loop(0, W)
        def _(r):
            @pl.loop(0, 128, step=16)
            def _(i):
                rowsa[r, pl.ds(i, 16)] = zeros16

        @pl.loop(0, ROWS_PER_TILE // W)
        def _(j):
            off = sid * ROWS_PER_TILE + j * W
            pltpu.sync_copy(rowsa, acc.at[pl.ds(off, W)])

    zero_acc()
    if with_deg:
        @pl.when(cid == 0)
        def _():
            @pl.loop(0, ROWS_PER_TILE // W)
            def _(j):
                off = sid * ROWS_PER_TILE + j * W
                pltpu.sync_copy(rowsa.at[0], deg_acc.at[pl.ds(off, W)])
    plsc.subcore_barrier()

    def load_idx(w, srcv, dstv, gidxv, chunk):
        base = w * W
        pltpu.sync_copy(src_hbm.at[pl.ds(base, W)], srcv)
        pltpu.sync_copy(dst_hbm.at[pl.ds(base, W)], dstv)

        @pl.loop(0, W, step=16)
        def _(i):
            gidxv[pl.ds(i, 16)] = srcv[pl.ds(i, 16)] * K + chunk

    def start_gather(gidxv, rows, sem):
        pltpu.async_copy(table.at[gidxv], rows, sem)

    def wait_gather(gidxv, rows, sem):
        pltpu.make_async_copy(table.at[gidxv], rows, sem).wait()

    def scatter(kk, rows, dstv):
        pltpu.sync_copy(rows, acc.at[dstv], add=True)
        if with_deg and kk == 0:
            @pl.when(cid == 0)
            def _():
                pltpu.sync_copy(onesv, deg_acc.at[dstv], add=True)

    npairs = (NWIN // NSUB + 2) // 2

    for kk in range(passes):
        chunk = cid * passes + kk

        load_idx(sid, srcva, dstva, gidxva, chunk)
        start_gather(gidxva, rowsa, sema)

        @pl.loop(0, npairs)
        def _(p):
            wa = sid + p * 2 * NSUB
            wb = wa + NSUB
            wa2 = wa + 2 * NSUB

            @pl.when(wb < NWIN)
            def _():
                load_idx(wb, srcvb, dstvb, gidxvb, chunk)

            @pl.when(wa < NWIN)
            def _():
                wait_gather(gidxva, rowsa, sema)

            @pl.when(wb < NWIN)
            def _():
                start_gather(gidxvb, rowsb, semb)

            @pl.when(wa < NWIN)
            def _():
                scatter(kk, rowsa, dstva)

            @pl.when(wa2 < NWIN)
            def _():
                load_idx(wa2, srcva, dstva, gidxva, chunk)

            @pl.when(wb < NWIN)
            def _():
                wait_gather(gidxvb, rowsb, semb)

            @pl.when(wa2 < NWIN)
            def _():
                start_gather(gidxva, rowsa, sema)

            @pl.when(wb < NWIN)
            def _():
                scatter(kk, rowsb, dstvb)

        plsc.subcore_barrier()

        @pl.loop(0, ROWS_PER_TILE // W)
        def _(j):
            off = sid * ROWS_PER_TILE + j * W
            pltpu.sync_copy(acc.at[pl.ds(off, W)],
                            agg_out.at[chunk, pl.ds(off, W)])
        if with_deg and kk == 0:
            @pl.when(cid == 0)
            def _():
                @pl.loop(0, ROWS_PER_TILE // W)
                def _(j):
                    off = sid * ROWS_PER_TILE + j * W
                    pltpu.sync_copy(deg_acc.at[pl.ds(off, W)],
                                    deg_out.at[pl.ds(off, W)])

        if kk + 1 < passes:
            zero_acc()
            plsc.subcore_barrier()


def _sc_agg(table, src, dst, K, with_deg):
    mesh = plsc.VectorSubcoreMesh(core_axis_name="c", subcore_axis_name="s")
    out_type = [jax.ShapeDtypeStruct((K, NP, 128), jnp.float32)]
    if with_deg:
        out_type.append(jax.ShapeDtypeStruct((NP,), jnp.float32))
    scratch = [
        pltpu.VMEM_SHARED((NP, 128), jnp.float32),
        pltpu.VMEM_SHARED((NP,), jnp.float32),
        pltpu.VMEM((W,), jnp.int32),
        pltpu.VMEM((W,), jnp.int32),
        pltpu.VMEM((W,), jnp.int32),
        pltpu.VMEM((W,), jnp.int32),
        pltpu.VMEM((W,), jnp.int32),
        pltpu.VMEM((W,), jnp.int32),
        pltpu.VMEM((W, 128), jnp.float32),
        pltpu.VMEM((W, 128), jnp.float32),
        pltpu.VMEM((W,), jnp.float32),
        pltpu.SemaphoreType.DMA,
        pltpu.SemaphoreType.DMA,
    ]
    body = functools.partial(_sc_agg_body, K, with_deg)
    fn = pl.kernel(body, out_type=out_type, mesh=mesh, scratch_types=scratch,
                   name=f"sc_sage_agg_k{K}")
    return fn(table, src, dst)


_DOT = functools.partial(
    lax.dot_general,
    dimension_numbers=(((1,), (1,)), ((), ())),
    precision=lax.Precision.HIGHEST,
    preferred_element_type=jnp.float32,
)


def _tc_lin_r_body(x_ref, w_ref, o_ref):
    o_ref[...] = _DOT(x_ref[...], w_ref[...])


def _tc_lin_r(x, w):
    f = x.shape[1]
    h = w.shape[0]
    return pl.pallas_call(
        _tc_lin_r_body,
        grid=(GRID,),
        in_specs=[pl.BlockSpec((RB, f), lambda i: (i, 0)),
                  pl.BlockSpec((h, f), lambda i: (0, 0))],
        out_specs=pl.BlockSpec((RB, h), lambda i: (i, 0)),
        out_shape=jax.ShapeDtypeStruct((NP, h), jnp.float32),
    )(x, w)


def _tc_layer_body(nchunks, emit_next, agg_ref, recip_ref, xr_ref, wl_ref,
                   b_ref, *rest):
    if emit_next:
        wn_ref, h_ref, xrn_ref = rest
    else:
        wc_ref, bc_ref, o_ref = rest
    z = _DOT(agg_ref[0], wl_ref[:, pl.ds(0, 128)])
    for k in range(1, nchunks):
        z += _DOT(agg_ref[k], wl_ref[:, pl.ds(k * 128, 128)])
    z = z * recip_ref[...] + xr_ref[...] + b_ref[...]
    h = jnp.maximum(z, 0.0)
    if emit_next:
        h_ref[...] = h
        xrn_ref[...] = _DOT(h, wn_ref[...])
    else:
        o_ref[...] = _DOT(h, wc_ref[...]) + bc_ref[...]


def _tc_layer1(agg, recip, xr, wl, b, wn):
    return pl.pallas_call(
        functools.partial(_tc_layer_body, 2, True),
        grid=(GRID,),
        in_specs=[pl.BlockSpec((2, RB, 128), lambda i: (0, i, 0)),
                  pl.BlockSpec((RB, 1), lambda i: (i, 0)),
                  pl.BlockSpec((RB, H), lambda i: (i, 0)),
                  pl.BlockSpec((H, D), lambda i: (0, 0)),
                  pl.BlockSpec((1, H), lambda i: (0, 0)),
                  pl.BlockSpec((H, H), lambda i: (0, 0))],
        out_specs=[pl.BlockSpec((RB, H), lambda i: (i, 0)),
                   pl.BlockSpec((RB, H), lambda i: (i, 0))],
        out_shape=[jax.ShapeDtypeStruct((NP, H), jnp.float32),
                   jax.ShapeDtypeStruct((NP, H), jnp.float32)],
    )(agg, recip, xr, wl, b, wn)


def _tc_layer2(agg, recip, xr, wl, b, wc, bc):
    return pl.pallas_call(
        functools.partial(_tc_layer_body, 4, False),
        grid=(GRID,),
        in_specs=[pl.BlockSpec((4, RB, 128), lambda i: (0, i, 0)),
                  pl.BlockSpec((RB, 1), lambda i: (i, 0)),
                  pl.BlockSpec((RB, H), lambda i: (i, 0)),
                  pl.BlockSpec((H, H), lambda i: (0, 0)),
                  pl.BlockSpec((1, H), lambda i: (0, 0)),
                  pl.BlockSpec((C, H), lambda i: (0, 0)),
                  pl.BlockSpec((1, C), lambda i: (0, 0))],
        out_specs=pl.BlockSpec((RB, C), lambda i: (i, 0)),
        out_shape=jax.ShapeDtypeStruct((NP, C), jnp.float32),
    )(agg, recip, xr, wl, b, wc, bc)


def kernel(x, edge_index, W1_l, b1_l, W1_r, W2_l, b2_l, W2_r, Wc, bc):
    src = edge_index[0]
    dst = edge_index[1]
    x_p = jnp.zeros((NP, D), jnp.float32).at[:N].set(x)

    agg1, deg = _sc_agg(x_p.reshape(2 * NP, 128), src, dst, 2, True)
    recip = (1.0 / jnp.clip(deg, 1.0)).reshape(NP, 1)

    xr1 = _tc_lin_r(x_p, W1_r)
    h1, xr2 = _tc_layer1(agg1, recip, xr1, W1_l, b1_l.reshape(1, H), W2_r)

    (agg2,) = _sc_agg(h1.reshape(4 * NP, 128), src, dst, 4, False)

    out = _tc_layer2(agg2, recip, xr2, W2_l, b2_l.reshape(1, H),
                     Wc, bc.reshape(1, C))
    return out[:N]


# speedup vs baseline: 5.1073x; 1.0453x over previous
import functools

import jax
import jax.numpy as jnp
from jax import lax
from jax.experimental import pallas as pl
from jax.experimental.pallas import tpu as pltpu
from jax.experimental.pallas import tpu_sc as plsc

N = 10000
E = 160000
D = 256
H = 512
C = 16

NP = 10240
W = 128
NWIN = E // W
NSUB = 16
ROWS_PER_TILE = NP // NSUB
RB = 512
GRID = NP // RB


def _sc_agg_body(K, with_deg, *refs):
    if with_deg:
        (table, src_hbm, dst_hbm, agg_out, deg_out,
         acc, deg_acc, srcva, dstva, gidxva, srcvb, dstvb, gidxvb,
         rowsa, rowsb, onesv, sema, semb) = refs
    else:
        (table, src_hbm, dst_hbm, agg_out,
         acc, deg_acc, srcva, dstva, gidxva, srcvb, dstvb, gidxvb,
         rowsa, rowsb, onesv, sema, semb) = refs
        deg_out = None

    cid = lax.axis_index("c")
    sid = lax.axis_index("s")
    passes = K // 2

    zeros16 = jnp.zeros((16,), jnp.float32)
    ones16 = jnp.ones((16,), jnp.float32)

    @pl.loop(0, W, step=16)
    def _(i):
        onesv[pl.ds(i, 16)] = ones16

    def zero_acc():
        @pl.loop(0, W)
        def _(r):
            @pl.loop(0, 128, step=16)
            def _(i):
                rowsa[r, pl.ds(i, 16)] = zeros16

        @pl.loop(0, ROWS_PER_TILE // W)
        def _(j):
            off = sid * ROWS_PER_TILE + j * W
            pltpu.sync_copy(rowsa, acc.at[pl.ds(off, W)])

    zero_acc()
    if with_deg:
        @pl.when(cid == 0)
        def _():
            @pl.loop(0, ROWS_PER_TILE // W)
            def _(j):
                off = sid * ROWS_PER_TILE + j * W
                pltpu.sync_copy(rowsa.at[0], deg_acc.at[pl.ds(off, W)])
    plsc.subcore_barrier()

    def load_idx(w, srcv, dstv, gidxv, chunk):
        base = w * W
        pltpu.sync_copy(src_hbm.at[pl.ds(base, W)], srcv)
        pltpu.sync_copy(dst_hbm.at[pl.ds(base, W)], dstv)

        @pl.loop(0, W, step=16)
        def _(i):
            gidxv[pl.ds(i, 16)] = srcv[pl.ds(i, 16)] * K + chunk

    def start_gather(gidxv, rows, sem):
        pltpu.async_copy(table.at[gidxv], rows, sem)

    def wait_gather(gidxv, rows, sem):
        pltpu.make_async_copy(table.at[gidxv], rows, sem).wait()

    def scatter(kk, rows, dstv):
        pltpu.sync_copy(rows, acc.at[dstv], add=True)
        if with_deg and kk == 0:
            @pl.when(cid == 0)
            def _():
                pltpu.sync_copy(onesv, deg_acc.at[dstv], add=True)

    npairs = (NWIN // NSUB + 2) // 2

    for kk in range(passes):
        chunk = cid * passes + kk

        load_idx(sid, srcva, dstva, gidxva, chunk)
        start_gather(gidxva, rowsa, sema)

        @pl.loop(0, npairs)
        def _(p):
            wa = sid + p * 2 * NSUB
            wb = wa + NSUB
            wa2 = wa + 2 * NSUB

            @pl.when(wb < NWIN)
            def _():
                load_idx(wb, srcvb, dstvb, gidxvb, chunk)

            @pl.when(wa < NWIN)
            def _():
                wait_gather(gidxva, rowsa, sema)

            @pl.when(wb < NWIN)
            def _():
                start_gather(gidxvb, rowsb, semb)

            @pl.when(wa < NWIN)
            def _():
                scatter(kk, rowsa, dstva)

            @pl.when(wa2 < NWIN)
            def _():
                load_idx(wa2, srcva, dstva, gidxva, chunk)

            @pl.when(wb < NWIN)
            def _():
                wait_gather(gidxvb, rowsb, semb)

            @pl.when(wa2 < NWIN)
            def _():
                start_gather(gidxva, rowsa, sema)

            @pl.when(wb < NWIN)
            def _():
                scatter(kk, rowsb, dstvb)

        plsc.subcore_barrier()

        @pl.loop(0, ROWS_PER_TILE // W)
        def _(j):
            off = sid * ROWS_PER_TILE + j * W
            pltpu.sync_copy(acc.at[pl.ds(off, W)],
                            agg_out.at[chunk, pl.ds(off, W)])
        if with_deg and kk == 0:
            @pl.when(cid == 0)
            def _():
                @pl.loop(0, ROWS_PER_TILE // W)
                def _(j):
                    off = sid * ROWS_PER_TILE + j * W
                    pltpu.sync_copy(deg_acc.at[pl.ds(off, W)],
                                    deg_out.at[pl.ds(off, W)])

        if kk + 1 < passes:
            zero_acc()
            plsc.subcore_barrier()


def _sc_agg(table, src, dst, K, with_deg):
    mesh = plsc.VectorSubcoreMesh(core_axis_name="c", subcore_axis_name="s")
    out_type = [jax.ShapeDtypeStruct((K, NP, 128), jnp.float32)]
    if with_deg:
        out_type.append(jax.ShapeDtypeStruct((NP,), jnp.float32))
    scratch = [
        pltpu.VMEM_SHARED((NP, 128), jnp.float32),
        pltpu.VMEM_SHARED((NP,), jnp.float32),
        pltpu.VMEM((W,), jnp.int32),
        pltpu.VMEM((W,), jnp.int32),
        pltpu.VMEM((W,), jnp.int32),
        pltpu.VMEM((W,), jnp.int32),
        pltpu.VMEM((W,), jnp.int32),
        pltpu.VMEM((W,), jnp.int32),
        pltpu.VMEM((W, 128), jnp.float32),
        pltpu.VMEM((W, 128), jnp.float32),
        pltpu.VMEM((W,), jnp.float32),
        pltpu.SemaphoreType.DMA,
        pltpu.SemaphoreType.DMA,
    ]
    body = functools.partial(_sc_agg_body, K, with_deg)
    fn = pl.kernel(body, out_type=out_type, mesh=mesh, scratch_types=scratch,
                   name=f"sc_sage_agg_k{K}")
    return fn(table, src, dst)


_DOT = functools.partial(
    lax.dot_general,
    dimension_numbers=(((1,), (1,)), ((), ())),
    precision=lax.Precision.HIGHEST,
    preferred_element_type=jnp.float32,
)


def _tc_lin_r_body(x_ref, w_ref, o_ref):
    o_ref[...] = _DOT(x_ref[...], w_ref[...])


def _tc_lin_r(x, w):
    f = x.shape[1]
    h = w.shape[0]
    return pl.pallas_call(
        _tc_lin_r_body,
        grid=(GRID,),
        in_specs=[pl.BlockSpec((RB, f), lambda i: (i, 0)),
                  pl.BlockSpec((h, f), lambda i: (0, 0))],
        out_specs=pl.BlockSpec((RB, h), lambda i: (i, 0)),
        out_shape=jax.ShapeDtypeStruct((NP, h), jnp.float32),
    )(x, w)


def _tc_layer_body(nchunks, emit_next, agg_ref, recip_ref, xr_ref, wl_ref,
                   b_ref, *rest):
    if emit_next:
        (h_ref,) = rest
    else:
        wc_ref, bc_ref, o_ref = rest
    z = _DOT(agg_ref[0], wl_ref[:, pl.ds(0, 128)])
    for k in range(1, nchunks):
        z += _DOT(agg_ref[k], wl_ref[:, pl.ds(k * 128, 128)])
    z = z * recip_ref[...] + xr_ref[...] + b_ref[...]
    h = jnp.maximum(z, 0.0)
    if emit_next:
        h_ref[...] = h
    else:
        o_ref[...] = _DOT(h, wc_ref[...]) + bc_ref[...]


def _tc_layer1(agg, recip, xr, wl, b):
    return pl.pallas_call(
        functools.partial(_tc_layer_body, 2, True),
        grid=(GRID,),
        in_specs=[pl.BlockSpec((2, RB, 128), lambda i: (0, i, 0)),
                  pl.BlockSpec((RB, 1), lambda i: (i, 0)),
                  pl.BlockSpec((RB, H), lambda i: (i, 0)),
                  pl.BlockSpec((H, D), lambda i: (0, 0)),
                  pl.BlockSpec((1, H), lambda i: (0, 0))],
        out_specs=pl.BlockSpec((RB, H), lambda i: (i, 0)),
        out_shape=jax.ShapeDtypeStruct((NP, H), jnp.float32),
    )(agg, recip, xr, wl, b)


def _tc_layer2(agg, recip, xr, wl, b, wc, bc):
    return pl.pallas_call(
        functools.partial(_tc_layer_body, 4, False),
        grid=(GRID,),
        in_specs=[pl.BlockSpec((4, RB, 128), lambda i: (0, i, 0)),
                  pl.BlockSpec((RB, 1), lambda i: (i, 0)),
                  pl.BlockSpec((RB, H), lambda i: (i, 0)),
                  pl.BlockSpec((H, H), lambda i: (0, 0)),
                  pl.BlockSpec((1, H), lambda i: (0, 0)),
                  pl.BlockSpec((C, H), lambda i: (0, 0)),
                  pl.BlockSpec((1, C), lambda i: (0, 0))],
        out_specs=pl.BlockSpec((RB, C), lambda i: (i, 0)),
        out_shape=jax.ShapeDtypeStruct((NP, C), jnp.float32),
    )(agg, recip, xr, wl, b, wc, bc)


def kernel(x, edge_index, W1_l, b1_l, W1_r, W2_l, b2_l, W2_r, Wc, bc):
    src = edge_index[0]
    dst = edge_index[1]
    x_p = jnp.zeros((NP, D), jnp.float32).at[:N].set(x)

    agg1, deg = _sc_agg(x_p.reshape(2 * NP, 128), src, dst, 2, True)
    recip = (1.0 / jnp.clip(deg, 1.0)).reshape(NP, 1)

    xr1 = _tc_lin_r(x_p, W1_r)
    h1 = _tc_layer1(agg1, recip, xr1, W1_l, b1_l.reshape(1, H))

    (agg2,) = _sc_agg(h1.reshape(4 * NP, 128), src, dst, 4, False)
    xr2 = _tc_lin_r(h1, W2_r)

    out = _tc_layer2(agg2, recip, xr2, W2_l, b2_l.reshape(1, H),
                     Wc, bc.reshape(1, C))
    return out[:N]


# speedup vs baseline: 6.0432x; 1.1832x over previous
import functools

import jax
import jax.numpy as jnp
from jax import lax
from jax.experimental import pallas as pl
from jax.experimental.pallas import tpu as pltpu
from jax.experimental.pallas import tpu_sc as plsc

N = 10000
E = 160000
D = 256
H = 512
C = 16

NP = 10240
W = 128
NWIN = E // W
NSUB = 16
ROWS_PER_TILE = NP // NSUB
RB = 512
GRID = NP // RB


def _sc_agg_body(K, with_deg, chunk_major, *refs):
    if with_deg:
        (table, src_hbm, dst_hbm, agg_out, deg_out,
         acc, deg_acc, srcva, dstva, gidxva, srcvb, dstvb, gidxvb,
         rowsa, rowsb, onesv, sema, semb) = refs
    else:
        (table, src_hbm, dst_hbm, agg_out,
         acc, deg_acc, srcva, dstva, gidxva, srcvb, dstvb, gidxvb,
         rowsa, rowsb, onesv, sema, semb) = refs
        deg_out = None

    cid = lax.axis_index("c")
    sid = lax.axis_index("s")
    passes = K // 2

    zeros16 = jnp.zeros((16,), jnp.float32)
    ones16 = jnp.ones((16,), jnp.float32)

    @pl.loop(0, W, step=16)
    def _(i):
        onesv[pl.ds(i, 16)] = ones16

    def zero_acc():
        @pl.loop(0, W)
        def _(r):
            @pl.loop(0, 128, step=16)
            def _(i):
                rowsa[r, pl.ds(i, 16)] = zeros16

        @pl.loop(0, ROWS_PER_TILE // W)
        def _(j):
            off = sid * ROWS_PER_TILE + j * W
            pltpu.sync_copy(rowsa, acc.at[pl.ds(off, W)])

    zero_acc()
    if with_deg:
        @pl.when(cid == 0)
        def _():
            @pl.loop(0, ROWS_PER_TILE // W)
            def _(j):
                off = sid * ROWS_PER_TILE + j * W
                pltpu.sync_copy(rowsa.at[0], deg_acc.at[pl.ds(off, W)])
    plsc.subcore_barrier()

    def load_idx(w, srcv, dstv, gidxv, chunk):
        base = w * W
        pltpu.sync_copy(src_hbm.at[pl.ds(base, W)], srcv)
        pltpu.sync_copy(dst_hbm.at[pl.ds(base, W)], dstv)

        @pl.loop(0, W, step=16)
        def _(i):
            if chunk_major:
                gidxv[pl.ds(i, 16)] = srcv[pl.ds(i, 16)] + chunk * NP
            else:
                gidxv[pl.ds(i, 16)] = srcv[pl.ds(i, 16)] * K + chunk

    def start_gather(gidxv, rows, sem):
        pltpu.async_copy(table.at[gidxv], rows, sem)

    def wait_gather(gidxv, rows, sem):
        pltpu.make_async_copy(table.at[gidxv], rows, sem).wait()

    def scatter(kk, rows, dstv):
        pltpu.sync_copy(rows, acc.at[dstv], add=True)
        if with_deg and kk == 0:
            @pl.when(cid == 0)
            def _():
                pltpu.sync_copy(onesv, deg_acc.at[dstv], add=True)

    npairs = (NWIN // NSUB + 2) // 2

    for kk in range(passes):
        chunk = cid * passes + kk

        load_idx(sid, srcva, dstva, gidxva, chunk)
        start_gather(gidxva, rowsa, sema)

        @pl.loop(0, npairs)
        def _(p):
            wa = sid + p * 2 * NSUB
            wb = wa + NSUB
            wa2 = wa + 2 * NSUB

            @pl.when(wb < NWIN)
            def _():
                load_idx(wb, srcvb, dstvb, gidxvb, chunk)

            @pl.when(wa < NWIN)
            def _():
                wait_gather(gidxva, rowsa, sema)

            @pl.when(wb < NWIN)
            def _():
                start_gather(gidxvb, rowsb, semb)

            @pl.when(wa < NWIN)
            def _():
                scatter(kk, rowsa, dstva)

            @pl.when(wa2 < NWIN)
            def _():
                load_idx(wa2, srcva, dstva, gidxva, chunk)

            @pl.when(wb < NWIN)
            def _():
                wait_gather(gidxvb, rowsb, semb)

            @pl.when(wa2 < NWIN)
            def _():
                start_gather(gidxva, rowsa, sema)

            @pl.when(wb < NWIN)
            def _():
                scatter(kk, rowsb, dstvb)

        plsc.subcore_barrier()

        @pl.loop(0, ROWS_PER_TILE // W)
        def _(j):
            off = sid * ROWS_PER_TILE + j * W
            pltpu.sync_copy(acc.at[pl.ds(off, W)],
                            agg_out.at[chunk, pl.ds(off, W)])
        if with_deg and kk == 0:
            @pl.when(cid == 0)
            def _():
                @pl.loop(0, ROWS_PER_TILE // W)
                def _(j):
                    off = sid * ROWS_PER_TILE + j * W
                    pltpu.sync_copy(deg_acc.at[pl.ds(off, W)],
                                    deg_out.at[pl.ds(off, W)])

        if kk + 1 < passes:
            zero_acc()
            plsc.subcore_barrier()


def _sc_agg(table, src, dst, K, with_deg, chunk_major=False):
    mesh = plsc.VectorSubcoreMesh(core_axis_name="c", subcore_axis_name="s")
    out_type = [jax.ShapeDtypeStruct((K, NP, 128), jnp.float32)]
    if with_deg:
        out_type.append(jax.ShapeDtypeStruct((NP,), jnp.float32))
    scratch = [
        pltpu.VMEM_SHARED((NP, 128), jnp.float32),
        pltpu.VMEM_SHARED((NP,), jnp.float32),
        pltpu.VMEM((W,), jnp.int32),
        pltpu.VMEM((W,), jnp.int32),
        pltpu.VMEM((W,), jnp.int32),
        pltpu.VMEM((W,), jnp.int32),
        pltpu.VMEM((W,), jnp.int32),
        pltpu.VMEM((W,), jnp.int32),
        pltpu.VMEM((W, 128), jnp.float32),
        pltpu.VMEM((W, 128), jnp.float32),
        pltpu.VMEM((W,), jnp.float32),
        pltpu.SemaphoreType.DMA,
        pltpu.SemaphoreType.DMA,
    ]
    body = functools.partial(_sc_agg_body, K, with_deg, chunk_major)
    fn = pl.kernel(body, out_type=out_type, mesh=mesh, scratch_types=scratch,
                   name=f"sc_sage_agg_k{K}")
    return fn(table, src, dst)


_DOT = functools.partial(
    lax.dot_general,
    dimension_numbers=(((1,), (1,)), ((), ())),
    preferred_element_type=jnp.float32,
)


def _dot3(a, b):
    a_hi = a.astype(jnp.bfloat16)
    a_lo = (a - a_hi.astype(jnp.float32)).astype(jnp.bfloat16)
    b_hi = b.astype(jnp.bfloat16)
    b_lo = (b - b_hi.astype(jnp.float32)).astype(jnp.bfloat16)
    return _DOT(a_hi, b_hi) + (_DOT(a_hi, b_lo) + _DOT(a_lo, b_hi))


def _tc_lin_r_body(x_ref, w_ref, o_ref):
    o_ref[...] = _dot3(x_ref[...], w_ref[...])


def _tc_lin_r(x, w):
    f = x.shape[1]
    h = w.shape[0]
    return pl.pallas_call(
        _tc_lin_r_body,
        grid=(GRID,),
        in_specs=[pl.BlockSpec((RB, f), lambda i: (i, 0)),
                  pl.BlockSpec((h, f), lambda i: (0, 0))],
        out_specs=pl.BlockSpec((RB, h), lambda i: (i, 0)),
        out_shape=jax.ShapeDtypeStruct((NP, h), jnp.float32),
    )(x, w)


def _tc_lin_rc_body(hc_ref, w_ref, o_ref):
    a = jnp.concatenate([hc_ref[k] for k in range(hc_ref.shape[0])], axis=1)
    o_ref[...] = _dot3(a, w_ref[...])


def _tc_lin_rc(hc, w):
    k = hc.shape[0]
    h = w.shape[0]
    return pl.pallas_call(
        _tc_lin_rc_body,
        grid=(GRID,),
        in_specs=[pl.BlockSpec((k, RB, 128), lambda i: (0, i, 0)),
                  pl.BlockSpec((h, k * 128), lambda i: (0, 0))],
        out_specs=pl.BlockSpec((RB, h), lambda i: (i, 0)),
        out_shape=jax.ShapeDtypeStruct((NP, h), jnp.float32),
    )(hc, w)


def _tc_layer_body(nchunks, emit_next, agg_ref, recip_ref, xr_ref, wl_ref,
                   b_ref, *rest):
    if emit_next:
        (hc_ref,) = rest
    else:
        wc_ref, bc_ref, o_ref = rest
    a = jnp.concatenate([agg_ref[k] for k in range(nchunks)], axis=1)
    z = _dot3(a, wl_ref[...])
    z = z * recip_ref[...] + xr_ref[...] + b_ref[...]
    h = jnp.maximum(z, 0.0)
    if emit_next:
        for k in range(4):
            hc_ref[k] = h[:, k * 128:(k + 1) * 128]
    else:
        o_ref[...] = _dot3(h, wc_ref[...]) + bc_ref[...]


def _tc_layer1(agg, recip, xr, wl, b):
    return pl.pallas_call(
        functools.partial(_tc_layer_body, 2, True),
        grid=(GRID,),
        in_specs=[pl.BlockSpec((2, RB, 128), lambda i: (0, i, 0)),
                  pl.BlockSpec((RB, 1), lambda i: (i, 0)),
                  pl.BlockSpec((RB, H), lambda i: (i, 0)),
                  pl.BlockSpec((H, D), lambda i: (0, 0)),
                  pl.BlockSpec((1, H), lambda i: (0, 0))],
        out_specs=pl.BlockSpec((4, RB, 128), lambda i: (0, i, 0)),
        out_shape=jax.ShapeDtypeStruct((4, NP, 128), jnp.float32),
    )(agg, recip, xr, wl, b)


def _tc_layer2(agg, recip, xr, wl, b, wc, bc):
    return pl.pallas_call(
        functools.partial(_tc_layer_body, 4, False),
        grid=(GRID,),
        in_specs=[pl.BlockSpec((4, RB, 128), lambda i: (0, i, 0)),
                  pl.BlockSpec((RB, 1), lambda i: (i, 0)),
                  pl.BlockSpec((RB, H), lambda i: (i, 0)),
                  pl.BlockSpec((H, H), lambda i: (0, 0)),
                  pl.BlockSpec((1, H), lambda i: (0, 0)),
                  pl.BlockSpec((C, H), lambda i: (0, 0)),
                  pl.BlockSpec((1, C), lambda i: (0, 0))],
        out_specs=pl.BlockSpec((RB, C), lambda i: (i, 0)),
        out_shape=jax.ShapeDtypeStruct((NP, C), jnp.float32),
    )(agg, recip, xr, wl, b, wc, bc)


def kernel(x, edge_index, W1_l, b1_l, W1_r, W2_l, b2_l, W2_r, Wc, bc):
    src = edge_index[0]
    dst = edge_index[1]
    x_p = jnp.zeros((NP, D), jnp.float32).at[:N].set(x)

    agg1, deg = _sc_agg(x_p.reshape(2 * NP, 128), src, dst, 2, True)
    recip = (1.0 / jnp.clip(deg, 1.0)).reshape(NP, 1)

    xr1 = _tc_lin_r(x_p, W1_r)
    h1c = _tc_layer1(agg1, recip, xr1, W1_l, b1_l.reshape(1, H))

    (agg2,) = _sc_agg(h1c.reshape(4 * NP, 128), src, dst, 4, False,
                      chunk_major=True)
    xr2 = _tc_lin_rc(h1c, W2_r)

    out = _tc_layer2(agg2, recip, xr2, W2_l, b2_l.reshape(1, H),
                     Wc, bc.reshape(1, C))
    return out[:N]


# speedup vs baseline: 6.9816x; 1.1553x over previous
import functools

import jax
import jax.numpy as jnp
from jax import lax
from jax.experimental import pallas as pl
from jax.experimental.pallas import tpu as pltpu
from jax.experimental.pallas import tpu_sc as plsc

N = 10000
E = 160000
D = 256
H = 512
C = 16

NP = 10240
W = 128
NWIN = E // W
NSUB = 16
ROWS_PER_TILE = NP // NSUB
RB = 512
GRID = NP // RB


def _sc_agg_body(K, with_deg, chunk_major, *refs):
    if with_deg:
        (table, src_hbm, dst_hbm, agg_out, deg_out,
         acc, deg_acc, srcva, dstva, gidxva, srcvb, dstvb, gidxvb,
         rowsa, rowsb, onesv, sema, semb, semia, semib) = refs
    else:
        (table, src_hbm, dst_hbm, agg_out,
         acc, deg_acc, srcva, dstva, gidxva, srcvb, dstvb, gidxvb,
         rowsa, rowsb, onesv, sema, semb, semia, semib) = refs
        deg_out = None

    cid = lax.axis_index("c")
    sid = lax.axis_index("s")
    passes = K // 2

    zeros16 = jnp.zeros((16,), jnp.float32)
    ones16 = jnp.ones((16,), jnp.float32)

    @pl.loop(0, W, step=16)
    def _(i):
        onesv[pl.ds(i, 16)] = ones16

    def zero_acc():
        @pl.loop(0, W)
        def _(r):
            @pl.loop(0, 128, step=16)
            def _(i):
                rowsa[r, pl.ds(i, 16)] = zeros16

        @pl.loop(0, ROWS_PER_TILE // W)
        def _(j):
            off = sid * ROWS_PER_TILE + j * W
            pltpu.sync_copy(rowsa, acc.at[pl.ds(off, W)])

    zero_acc()
    if with_deg:
        @pl.when(cid == 0)
        def _():
            @pl.loop(0, ROWS_PER_TILE // W)
            def _(j):
                off = sid * ROWS_PER_TILE + j * W
                pltpu.sync_copy(rowsa.at[0], deg_acc.at[pl.ds(off, W)])
    plsc.subcore_barrier()

    def start_idx(w, srcv, dstv, sem):
        base = w * W
        pltpu.async_copy(src_hbm.at[pl.ds(base, W)], srcv, sem)
        pltpu.async_copy(dst_hbm.at[pl.ds(base, W)], dstv, sem)

    def finish_idx(w, srcv, dstv, gidxv, sem, chunk):
        base = w * W
        pltpu.make_async_copy(src_hbm.at[pl.ds(base, W)], srcv, sem).wait()
        pltpu.make_async_copy(dst_hbm.at[pl.ds(base, W)], dstv, sem).wait()

        @pl.loop(0, W, step=16)
        def _(i):
            if chunk_major:
                gidxv[pl.ds(i, 16)] = srcv[pl.ds(i, 16)] + chunk * NP
            else:
                gidxv[pl.ds(i, 16)] = srcv[pl.ds(i, 16)] * K + chunk

    def start_gather(gidxv, rows, sem):
        pltpu.async_copy(table.at[gidxv], rows, sem)

    def wait_gather(gidxv, rows, sem):
        pltpu.make_async_copy(table.at[gidxv], rows, sem).wait()

    def scatter(kk, rows, dstv):
        pltpu.sync_copy(rows, acc.at[dstv], add=True)
        if with_deg and kk == 0:
            @pl.when(cid == 0)
            def _():
                pltpu.sync_copy(onesv, deg_acc.at[dstv], add=True)

    npairs = (NWIN // NSUB + 2) // 2

    for kk in range(passes):
        chunk = cid * passes + kk

        start_idx(sid, srcva, dstva, semia)
        finish_idx(sid, srcva, dstva, gidxva, semia, chunk)
        start_gather(gidxva, rowsa, sema)
        start_idx(sid + NSUB, srcvb, dstvb, semib)

        @pl.loop(0, npairs)
        def _(p):
            wa = sid + p * 2 * NSUB
            wb = wa + NSUB
            wa2 = wa + 2 * NSUB
            wb2 = wb + 2 * NSUB

            @pl.when(wb < NWIN)
            def _():
                finish_idx(wb, srcvb, dstvb, gidxvb, semib, chunk)

            @pl.when(wa < NWIN)
            def _():
                wait_gather(gidxva, rowsa, sema)

            @pl.when(wb < NWIN)
            def _():
                start_gather(gidxvb, rowsb, semb)

            @pl.when(wa < NWIN)
            def _():
                scatter(kk, rowsa, dstva)

            @pl.when(wa2 < NWIN)
            def _():
                start_idx(wa2, srcva, dstva, semia)
                finish_idx(wa2, srcva, dstva, gidxva, semia, chunk)
                start_gather(gidxva, rowsa, sema)

            @pl.when(wb < NWIN)
            def _():
                wait_gather(gidxvb, rowsb, semb)

            @pl.when(wb < NWIN)
            def _():
                scatter(kk, rowsb, dstvb)

            @pl.when(wb2 < NWIN)
            def _():
                start_idx(wb2, srcvb, dstvb, semib)

        plsc.subcore_barrier()

        @pl.loop(0, ROWS_PER_TILE // W)
        def _(j):
            off = sid * ROWS_PER_TILE + j * W
            pltpu.sync_copy(acc.at[pl.ds(off, W)],
                            agg_out.at[chunk, pl.ds(off, W)])
        if with_deg and kk == 0:
            @pl.when(cid == 0)
            def _():
                @pl.loop(0, ROWS_PER_TILE // W)
                def _(j):
                    off = sid * ROWS_PER_TILE + j * W
                    pltpu.sync_copy(deg_acc.at[pl.ds(off, W)],
                                    deg_out.at[pl.ds(off, W)])

        if kk + 1 < passes:
            zero_acc()
            plsc.subcore_barrier()


def _sc_agg(table, src, dst, K, with_deg, chunk_major=False):
    mesh = plsc.VectorSubcoreMesh(core_axis_name="c", subcore_axis_name="s")
    out_type = [jax.ShapeDtypeStruct((K, NP, 128), jnp.float32)]
    if with_deg:
        out_type.append(jax.ShapeDtypeStruct((NP,), jnp.float32))
    scratch = [
        pltpu.VMEM_SHARED((NP, 128), jnp.float32),
        pltpu.VMEM_SHARED((NP,), jnp.float32),
        pltpu.VMEM((W,), jnp.int32),
        pltpu.VMEM((W,), jnp.int32),
        pltpu.VMEM((W,), jnp.int32),
        pltpu.VMEM((W,), jnp.int32),
        pltpu.VMEM((W,), jnp.int32),
        pltpu.VMEM((W,), jnp.int32),
        pltpu.VMEM((W, 128), jnp.float32),
        pltpu.VMEM((W, 128), jnp.float32),
        pltpu.VMEM((W,), jnp.float32),
        pltpu.SemaphoreType.DMA,
        pltpu.SemaphoreType.DMA,
        pltpu.SemaphoreType.DMA,
        pltpu.SemaphoreType.DMA,
    ]
    body = functools.partial(_sc_agg_body, K, with_deg, chunk_major)
    fn = pl.kernel(body, out_type=out_type, mesh=mesh, scratch_types=scratch,
                   name=f"sc_sage_agg_k{K}")
    return fn(table, src, dst)


_DOT = functools.partial(
    lax.dot_general,
    dimension_numbers=(((1,), (1,)), ((), ())),
    preferred_element_type=jnp.float32,
)


def _dot3(a, b):
    a_hi = a.astype(jnp.bfloat16)
    a_lo = (a - a_hi.astype(jnp.float32)).astype(jnp.bfloat16)
    b_hi = b.astype(jnp.bfloat16)
    b_lo = (b - b_hi.astype(jnp.float32)).astype(jnp.bfloat16)
    return _DOT(a_hi, b_hi) + (_DOT(a_hi, b_lo) + _DOT(a_lo, b_hi))


def _tc_lin_r_body(x_ref, w_ref, o_ref):
    o_ref[...] = _dot3(x_ref[...], w_ref[...])


def _tc_lin_r(x, w):
    f = x.shape[1]
    h = w.shape[0]
    return pl.pallas_call(
        _tc_lin_r_body,
        grid=(GRID,),
        in_specs=[pl.BlockSpec((RB, f), lambda i: (i, 0)),
                  pl.BlockSpec((h, f), lambda i: (0, 0))],
        out_specs=pl.BlockSpec((RB, h), lambda i: (i, 0)),
        out_shape=jax.ShapeDtypeStruct((NP, h), jnp.float32),
    )(x, w)


def _tc_lin_rc_body(hc_ref, w_ref, o_ref):
    a = jnp.concatenate([hc_ref[k] for k in range(hc_ref.shape[0])], axis=1)
    o_ref[...] = _dot3(a, w_ref[...])


def _tc_lin_rc(hc, w):
    k = hc.shape[0]
    h = w.shape[0]
    return pl.pallas_call(
        _tc_lin_rc_body,
        grid=(GRID,),
        in_specs=[pl.BlockSpec((k, RB, 128), lambda i: (0, i, 0)),
                  pl.BlockSpec((h, k * 128), lambda i: (0, 0))],
        out_specs=pl.BlockSpec((RB, h), lambda i: (i, 0)),
        out_shape=jax.ShapeDtypeStruct((NP, h), jnp.float32),
    )(hc, w)


def _tc_layer_body(nchunks, emit_next, agg_ref, recip_ref, xr_ref, wl_ref,
                   b_ref, *rest):
    if emit_next:
        (hc_ref,) = rest
    else:
        wc_ref, bc_ref, o_ref = rest
    a = jnp.concatenate([agg_ref[k] for k in range(nchunks)], axis=1)
    z = _dot3(a, wl_ref[...])
    z = z * recip_ref[...] + xr_ref[...] + b_ref[...]
    h = jnp.maximum(z, 0.0)
    if emit_next:
        for k in range(4):
            hc_ref[k] = h[:, k * 128:(k + 1) * 128]
    else:
        o_ref[...] = _dot3(h, wc_ref[...]) + bc_ref[...]


def _tc_layer1(agg, recip, xr, wl, b):
    return pl.pallas_call(
        functools.partial(_tc_layer_body, 2, True),
        grid=(GRID,),
        in_specs=[pl.BlockSpec((2, RB, 128), lambda i: (0, i, 0)),
                  pl.BlockSpec((RB, 1), lambda i: (i, 0)),
                  pl.BlockSpec((RB, H), lambda i: (i, 0)),
                  pl.BlockSpec((H, D), lambda i: (0, 0)),
                  pl.BlockSpec((1, H), lambda i: (0, 0))],
        out_specs=pl.BlockSpec((4, RB, 128), lambda i: (0, i, 0)),
        out_shape=jax.ShapeDtypeStruct((4, NP, 128), jnp.float32),
    )(agg, recip, xr, wl, b)


def _tc_layer2(agg, recip, xr, wl, b, wc, bc):
    return pl.pallas_call(
        functools.partial(_tc_layer_body, 4, False),
        grid=(GRID,),
        in_specs=[pl.BlockSpec((4, RB, 128), lambda i: (0, i, 0)),
                  pl.BlockSpec((RB, 1), lambda i: (i, 0)),
                  pl.BlockSpec((RB, H), lambda i: (i, 0)),
                  pl.BlockSpec((H, H), lambda i: (0, 0)),
                  pl.BlockSpec((1, H), lambda i: (0, 0)),
                  pl.BlockSpec((C, H), lambda i: (0, 0)),
                  pl.BlockSpec((1, C), lambda i: (0, 0))],
        out_specs=pl.BlockSpec((RB, C), lambda i: (i, 0)),
        out_shape=jax.ShapeDtypeStruct((NP, C), jnp.float32),
    )(agg, recip, xr, wl, b, wc, bc)


def kernel(x, edge_index, W1_l, b1_l, W1_r, W2_l, b2_l, W2_r, Wc, bc):
    src = edge_index[0]
    dst = edge_index[1]
    x_p = jnp.zeros((NP, D), jnp.float32).at[:N].set(x)

    agg1, deg = _sc_agg(x_p.reshape(2 * NP, 128), src, dst, 2, True)
    recip = (1.0 / jnp.clip(deg, 1.0)).reshape(NP, 1)

    xr1 = _tc_lin_r(x_p, W1_r)
    h1c = _tc_layer1(agg1, recip, xr1, W1_l, b1_l.reshape(1, H))

    (agg2,) = _sc_agg(h1c.reshape(4 * NP, 128), src, dst, 4, False,
                      chunk_major=True)
    xr2 = _tc_lin_rc(h1c, W2_r)

    out = _tc_layer2(agg2, recip, xr2, W2_l, b2_l.reshape(1, H),
                     Wc, bc.reshape(1, C))
    return out[:N]
